# Initial kernel scaffold; baseline (speedup 1.0000x reference)
import functools

import jax
import jax.numpy as jnp
from jax import lax
from jax.experimental import pallas as pl
from jax.experimental.pallas import tpu as pltpu
from jax.experimental.pallas import tpu_sc as plsc

_N = 10000
_D = 128
_E = 320000
_R = 4
_NB = 2
_L = 2

_B = 80
_ROWS = _E // _B
_HALF = _N // 2
_SLAB = 5120
_CNT_PAD = 40960
_DEG_PAD = 10240

_f32 = jnp.float32
_i32 = jnp.int32


def _mesh():
    return plsc.VectorSubcoreMesh(core_axis_name="c", subcore_axis_name="s")


def _stats_body(src2, dst2, et2, compt, zer1d, ones1d,
                w_out, deg_out,
                cnt_sp, deg_sp,
                src_v, dst_v, et_v, idx_v, ones_v, cnt_v, comp_v,
                w0_v, w1_v, w2_v, w3_v):
    cid = lax.axis_index("c")
    sid = lax.axis_index("s")

    pltpu.sync_copy(zer1d.at[pl.ds(0, _CNT_PAD // 16)],
                    cnt_sp.at[pl.ds(sid * (_CNT_PAD // 16), _CNT_PAD // 16)])
    pltpu.sync_copy(zer1d.at[pl.ds(0, _DEG_PAD // 16)],
                    deg_sp.at[pl.ds(sid * (_DEG_PAD // 16), _DEG_PAD // 16)])
    pltpu.sync_copy(ones1d, ones_v)
    pltpu.sync_copy(compt, comp_v)
    plsc.subcore_barrier()

    rows_per_tile = _ROWS // 16
    blocks = rows_per_tile // 25

    def count_block(blk, _):
        r0 = sid * rows_per_tile + blk * 25
        pltpu.sync_copy(src2.at[pl.ds(r0, 25)], src_v)
        pltpu.sync_copy(dst2.at[pl.ds(r0, 25)], dst_v)
        pltpu.sync_copy(et2.at[pl.ds(r0, 25)], et_v)
        for j in range(25):
            for c in range(5):
                sl = pl.ds(c * 16, 16)
                idx_v[j, sl] = et_v[j, sl] * _N + dst_v[j, sl]
        for j in range(25):
            pltpu.sync_copy(ones_v, cnt_sp.at[idx_v.at[j]], add=True)
            pltpu.sync_copy(ones_v, deg_sp.at[src_v.at[j]], add=True)
        return 0

    lax.fori_loop(0, blocks, count_block, 0)
    plsc.subcore_barrier()

    @pl.when(jnp.logical_and(cid == 0, sid == 0))
    def _():
        pltpu.sync_copy(deg_sp.at[pl.ds(0, _N)], deg_out)

    pltpu.sync_copy(cnt_sp, cnt_v)
    wid = cid * 16 + sid
    rows_per_w = _ROWS // 32
    wblocks = rows_per_w // 25

    def weight_block(blk, _):
        r0 = wid * rows_per_w + blk * 25
        pltpu.sync_copy(dst2.at[pl.ds(r0, 25)], dst_v)
        pltpu.sync_copy(et2.at[pl.ds(r0, 25)], et_v)
        wbufs = (w0_v, w1_v, w2_v, w3_v)
        for j in range(25):
            for c in range(5):
                sl = pl.ds(c * 16, 16)
                t16 = et_v[j, sl]
                i4 = t16 * _N + dst_v[j, sl]
                ic = 1.0 / plsc.load_gather(cnt_v, [i4])
                for lb in range(4):
                    cw = plsc.load_gather(comp_v, [t16 + (lb * _R)])
                    wbufs[lb][j, sl] = cw * ic
        for lb in range(4):
            pltpu.sync_copy(wbufs[lb], w_out.at[lb, pl.ds(r0, 25)])
        return 0

    lax.fori_loop(0, wblocks, weight_block, 0)


def _graph_stats(src2, dst2, et2, compt, zer1d, ones1d):
    kfn = pl.kernel(
        _stats_body,
        out_type=(jax.ShapeDtypeStruct((4, _ROWS, _B), _f32),
                  jax.ShapeDtypeStruct((_N,), _f32)),
        mesh=_mesh(),
        scratch_types=[
            pltpu.VMEM_SHARED((_CNT_PAD,), _f32),
            pltpu.VMEM_SHARED((_DEG_PAD,), _f32),
            pltpu.VMEM((25, _B), _i32),
            pltpu.VMEM((25, _B), _i32),
            pltpu.VMEM((25, _B), _i32),
            pltpu.VMEM((25, _B), _i32),
            pltpu.VMEM((_B,), _f32),
            pltpu.VMEM((_CNT_PAD,), _f32),
            pltpu.VMEM((16,), _f32),
            pltpu.VMEM((25, _B), _f32),
            pltpu.VMEM((25, _B), _f32),
            pltpu.VMEM((25, _B), _f32),
            pltpu.VMEM((25, _B), _f32),
        ],
    )
    return kfn(src2, dst2, et2, compt, zer1d, ones1d)


def _edge_body(layer, h_hbm, src2, dst2, w_hbm, zer2d,
               z0_out, z1_out, num_out,
               z0_sp, z1_sp, num_sp,
               sidx_v, didx_v, zi_v, si_v, w0_v, w1_v,
               hs_v, hd_v, m0_v, m1_v, sem, sem2):
    cid = lax.axis_index("c")
    sid = lax.axis_index("s")
    base_node = cid * _HALF

    for q in range(5):
        r0 = sid * 320 + q * 64
        pltpu.sync_copy(zer2d.at[pl.ds(0, 64)], z0_sp.at[pl.ds(r0, 64)])
        pltpu.sync_copy(zer2d.at[pl.ds(0, 64)], z1_sp.at[pl.ds(r0, 64)])
        pltpu.sync_copy(zer2d.at[pl.ds(0, 64)], num_sp.at[pl.ds(r0, 64)])
    plsc.subcore_barrier()

    rows_per_tile = _ROWS // 16

    def edge_block(blk, _):
        row = sid * rows_per_tile + blk
        pltpu.sync_copy(src2.at[row], sidx_v)
        pltpu.sync_copy(dst2.at[row], didx_v)
        pltpu.sync_copy(w_hbm.at[2 * layer, row], w0_v)
        pltpu.sync_copy(w_hbm.at[2 * layer + 1, row], w1_v)
        cp1 = pltpu.async_copy(h_hbm.at[sidx_v], hs_v, sem)
        cp2 = pltpu.async_copy(h_hbm.at[didx_v], hd_v, sem2)
        cp1.wait()
        cp2.wait()
        for c in range(5):
            sl = pl.ds(c * 16, 16)
            dl = didx_v[sl] - base_node
            zi_v[sl] = jnp.where(
                jnp.logical_and(dl >= 0, dl < _HALF), dl, _HALF)
            sl2 = sidx_v[sl] - base_node
            si_v[sl] = jnp.where(
                jnp.logical_and(sl2 >= 0, sl2 < _HALF), sl2, _HALF)
        for r in range(_B):
            ridx = jnp.full((16,), r, _i32)
            w0b = plsc.load_gather(w0_v, [ridx])
            w1b = plsc.load_gather(w1_v, [ridx])
            for c in range(8):
                sl = pl.ds(c * 16, 16)
                hs = hs_v[r, sl]
                hd = hd_v[r, sl]
                df = hs - hd
                m0_v[r, sl] = hs * w0b
                m1_v[r, sl] = hs * w1b
                hd_v[r, sl] = df * df
        pltpu.sync_copy(m0_v, z0_sp.at[zi_v], add=True)
        pltpu.sync_copy(m1_v, z1_sp.at[zi_v], add=True)
        pltpu.sync_copy(hd_v, num_sp.at[si_v], add=True)
        return 0

    lax.fori_loop(0, rows_per_tile, edge_block, 0)
    plsc.subcore_barrier()

    r0 = sid * 312
    pltpu.sync_copy(z0_sp.at[pl.ds(r0, 312)],
                    z0_out.at[pl.ds(base_node + r0, 312)])
    pltpu.sync_copy(z1_sp.at[pl.ds(r0, 312)],
                    z1_out.at[pl.ds(base_node + r0, 312)])
    pltpu.sync_copy(num_sp.at[pl.ds(r0, 312)],
                    num_out.at[pl.ds(base_node + r0, 312)])

    @pl.when(sid == 0)
    def _():
        pltpu.sync_copy(z0_sp.at[pl.ds(4992, 8)],
                        z0_out.at[pl.ds(base_node + 4992, 8)])
        pltpu.sync_copy(z1_sp.at[pl.ds(4992, 8)],
                        z1_out.at[pl.ds(base_node + 4992, 8)])
        pltpu.sync_copy(num_sp.at[pl.ds(4992, 8)],
                        num_out.at[pl.ds(base_node + 4992, 8)])


def _edge_pass(layer, h, src2, dst2, w, zer2d):
    kfn = pl.kernel(
        functools.partial(_edge_body, layer),
        out_type=(jax.ShapeDtypeStruct((_N, _D), _f32),
                  jax.ShapeDtypeStruct((_N, _D), _f32),
                  jax.ShapeDtypeStruct((_N, _D), _f32)),
        mesh=_mesh(),
        scratch_types=[
            pltpu.VMEM_SHARED((_SLAB, _D), _f32),
            pltpu.VMEM_SHARED((_SLAB, _D), _f32),
            pltpu.VMEM_SHARED((_SLAB, _D), _f32),
            pltpu.VMEM((_B,), _i32),
            pltpu.VMEM((_B,), _i32),
            pltpu.VMEM((_B,), _i32),
            pltpu.VMEM((_B,), _i32),
            pltpu.VMEM((_B,), _f32),
            pltpu.VMEM((_B,), _f32),
            pltpu.VMEM((_B, _D), _f32),
            pltpu.VMEM((_B, _D), _f32),
            pltpu.VMEM((_B, _D), _f32),
            pltpu.VMEM((_B, _D), _f32),
            pltpu.SemaphoreType.DMA,
            pltpu.SemaphoreType.DMA,
        ],
    )
    return kfn(h, src2, dst2, w, zer2d)


_MB = 2000


def _dense_in(x, W, b2):
    def body(x_ref, w_ref, b_ref, o_ref):
        o_ref[...] = jnp.maximum(
            jnp.dot(x_ref[...], w_ref[...], preferred_element_type=_f32)
            + b_ref[...], 0.0)

    return pl.pallas_call(
        body,
        grid=(_N // _MB,),
        in_specs=[
            pl.BlockSpec((_MB, _D), lambda i: (i, 0)),
            pl.BlockSpec((_D, _D), lambda i: (0, 0)),
            pl.BlockSpec((1, _D), lambda i: (0, 0)),
        ],
        out_specs=pl.BlockSpec((_MB, _D), lambda i: (i, 0)),
        out_shape=jax.ShapeDtypeStruct((_N, _D), _f32),
    )(x, W, b2)


def _gate(h, z0, z1, num, deg2, root, bas0, bas1, bias2, Wout, bout2, final):
    def body(h_ref, z0_ref, z1_ref, num_ref, deg_ref, root_ref, b0_ref,
             b1_ref, bias_ref, wo_ref, bo_ref, hn_ref, o_ref):
        hv = h_ref[...]
        xc = (jnp.dot(hv, root_ref[...], preferred_element_type=_f32)
              + jnp.dot(z0_ref[...], b0_ref[...], preferred_element_type=_f32)
              + jnp.dot(z1_ref[...], b1_ref[...], preferred_element_type=_f32)
              + bias_ref[...])
        x_ = jnp.maximum(xc, 0.0)
        tau = jnp.tanh(num_ref[...] / jnp.maximum(deg_ref[...], 1.0))
        hn = (1.0 - tau) * hv + tau * x_
        hn_ref[...] = hn
        if final:
            o_ref[...] = (jnp.dot(hn, wo_ref[...],
                                  preferred_element_type=_f32) + bo_ref[...])
        else:
            o_ref[...] = jnp.zeros_like(o_ref)

    mat = pl.BlockSpec((_MB, _D), lambda i: (i, 0))
    w128 = pl.BlockSpec((_D, _D), lambda i: (0, 0))
    row1 = pl.BlockSpec((1, _D), lambda i: (0, 0))
    return pl.pallas_call(
        body,
        grid=(_N // _MB,),
        in_specs=[mat, mat, mat, mat,
                  pl.BlockSpec((_MB, 1), lambda i: (i, 0)),
                  w128, w128, w128, row1, w128, row1],
        out_specs=(mat, mat),
        out_shape=(jax.ShapeDtypeStruct((_N, _D), _f32),
                   jax.ShapeDtypeStruct((_N, _D), _f32)),
    )(h, z0, z1, num, deg2, root, bas0, bas1, bias2, Wout, bout2)


def kernel(x, edge_index, edge_type, W_in, b_in, rgcn_bases, rgcn_comp,
           rgcn_root, rgcn_bias, gg_bases, gg_comp, gg_root, gg_bias,
           W_out, b_out):
    src2 = edge_index[0].reshape(_ROWS, _B)
    dst2 = edge_index[1].reshape(_ROWS, _B)
    et2 = edge_type.reshape(_ROWS, _B)
    compt = jnp.transpose(rgcn_comp, (0, 2, 1)).reshape(_L * _NB * _R)
    zer1d = jnp.zeros((_CNT_PAD // 16,), _f32)
    ones1d = jnp.ones((_B,), _f32)
    zer2d = jnp.zeros((64, _D), _f32)

    w, deg = _graph_stats(src2, dst2, et2, compt, zer1d, ones1d)
    deg2 = deg.reshape(_N, 1)

    h = _dense_in(x, W_in, b_in.reshape(1, _D))
    latents = [h]
    out = None
    for l in range(_L):
        z0, z1, num = _edge_pass(l, h, src2, dst2, w, zer2d)
        h, out = _gate(h, z0, z1, num, deg2,
                       rgcn_root[l], rgcn_bases[l, 0], rgcn_bases[l, 1],
                       rgcn_bias[l].reshape(1, _D),
                       W_out, b_out.reshape(1, _D), final=(l == _L - 1))
        latents.append(h)
    return (out, *latents)


# speedup vs baseline: 1.6876x; 1.6876x over previous
import functools

import jax
import jax.numpy as jnp
from jax import lax
from jax.experimental import pallas as pl
from jax.experimental.pallas import tpu as pltpu
from jax.experimental.pallas import tpu_sc as plsc

_N = 10000
_D = 128
_E = 320000
_R = 4
_NB = 2
_L = 2

_B = 80
_ROWS = 4096
_EP = _ROWS * _B
_HALF = _N // 2
_SLAB = 5120
_CNT_PAD = 40960
_DEG_PAD = 10240

_f32 = jnp.float32
_i32 = jnp.int32

_SC_PARAMS = dict(
    mesh=plsc.VectorSubcoreMesh(core_axis_name="c", subcore_axis_name="s"),
    compiler_params=pltpu.CompilerParams(needs_layout_passes=False),
)


def _stats_body(src2, dst2, et2, compt, zer1d, ones1d,
                w_out, deg_out,
                cnt_sp, deg_sp,
                src_v, dst_v, et_v, idx_v, didx_v, ones_v, cnt_v, comp_v,
                w0_v, w1_v, w2_v, w3_v):
    cid = lax.axis_index("c")
    sid = lax.axis_index("s")

    pltpu.sync_copy(zer1d.at[pl.ds(0, _CNT_PAD // 16)],
                    cnt_sp.at[pl.ds(sid * (_CNT_PAD // 16), _CNT_PAD // 16)])
    pltpu.sync_copy(zer1d.at[pl.ds(0, _DEG_PAD // 16)],
                    deg_sp.at[pl.ds(sid * (_DEG_PAD // 16), _DEG_PAD // 16)])
    pltpu.sync_copy(ones1d, ones_v)
    pltpu.sync_copy(compt, comp_v)
    plsc.subcore_barrier()

    rows_per_tile = _ROWS // 16

    def count_block(blk, _):
        r0 = sid * rows_per_tile + blk * 32
        pltpu.sync_copy(src2.at[pl.ds(r0, 32)], src_v)
        pltpu.sync_copy(dst2.at[pl.ds(r0, 32)], dst_v)
        pltpu.sync_copy(et2.at[pl.ds(r0, 32)], et_v)

        def compute_idx(j, _2):
            for c in range(5):
                sl = pl.ds(c * 16, 16)
                t16 = et_v[j, sl]
                idx_v[j, sl] = t16 * _N + dst_v[j, sl]
                didx_v[j, sl] = jnp.where(t16 >= _R, _N, src_v[j, sl])
            return 0

        lax.fori_loop(0, 32, compute_idx, 0)

        def scatter_row(j, _2):
            pltpu.sync_copy(ones_v, cnt_sp.at[idx_v.at[j]], add=True)
            pltpu.sync_copy(ones_v, deg_sp.at[didx_v.at[j]], add=True)
            return 0

        lax.fori_loop(0, 32, scatter_row, 0)
        return 0

    lax.fori_loop(0, rows_per_tile // 32, count_block, 0)
    plsc.subcore_barrier()

    @pl.when(jnp.logical_and(cid == 0, sid == 0))
    def _():
        pltpu.sync_copy(deg_sp, deg_out)

    pltpu.sync_copy(cnt_sp, cnt_v)
    wid = cid * 16 + sid
    rows_per_w = _ROWS // 32

    def weight_block(blk, _):
        r0 = wid * rows_per_w + blk * 32
        pltpu.sync_copy(dst2.at[pl.ds(r0, 32)], dst_v)
        pltpu.sync_copy(et2.at[pl.ds(r0, 32)], et_v)
        wbufs = (w0_v, w1_v, w2_v, w3_v)

        def compute_w(j, _2):
            for c in range(5):
                sl = pl.ds(c * 16, 16)
                t16 = et_v[j, sl]
                i4 = t16 * _N + dst_v[j, sl]
                ic = 1.0 / plsc.load_gather(cnt_v, [i4])
                for lb in range(4):
                    cw = plsc.load_gather(comp_v, [t16 + (lb * 8)])
                    wbufs[lb][j, sl] = cw * ic
            return 0

        lax.fori_loop(0, 32, compute_w, 0)
        for lb in range(4):
            pltpu.sync_copy(wbufs[lb], w_out.at[lb, pl.ds(r0, 32)])
        return 0

    lax.fori_loop(0, rows_per_w // 32, weight_block, 0)


def _graph_stats(src2, dst2, et2, compt, zer1d, ones1d):
    kfn = pl.kernel(
        _stats_body,
        out_type=(jax.ShapeDtypeStruct((4, _ROWS, _B), _f32),
                  jax.ShapeDtypeStruct((_DEG_PAD,), _f32)),
        scratch_types=[
            pltpu.VMEM_SHARED((_CNT_PAD,), _f32),
            pltpu.VMEM_SHARED((_DEG_PAD,), _f32),
            pltpu.VMEM((32, _B), _i32),
            pltpu.VMEM((32, _B), _i32),
            pltpu.VMEM((32, _B), _i32),
            pltpu.VMEM((32, _B), _i32),
            pltpu.VMEM((32, _B), _i32),
            pltpu.VMEM((_B,), _f32),
            pltpu.VMEM((_CNT_PAD,), _f32),
            pltpu.VMEM((32,), _f32),
            pltpu.VMEM((32, _B), _f32),
            pltpu.VMEM((32, _B), _f32),
            pltpu.VMEM((32, _B), _f32),
            pltpu.VMEM((32, _B), _f32),
        ],
        **_SC_PARAMS,
    )
    return kfn(src2, dst2, et2, compt, zer1d, ones1d)


def _z_body(layer, h_hbm, src2, dst2, w_hbm, zer2d,
            z0_out, z1_out,
            acc_sp,
            sidx_v, didx_v, w0b_v, w1b_v, zi_v, zi1_v,
            hs_v, m0_v, m1_v, sem):
    cid = lax.axis_index("c")
    sid = lax.axis_index("s")
    base_node = cid * _HALF

    for q in range(10):
        r0 = sid * 640 + q * 64
        pltpu.sync_copy(zer2d.at[pl.ds(0, 64)], acc_sp.at[pl.ds(r0, 64)])
    plsc.subcore_barrier()

    rows_per_tile = _ROWS // 16

    def group(g, _):
        r0 = sid * rows_per_tile + g * 32
        pltpu.sync_copy(src2.at[pl.ds(r0, 32)], sidx_v)
        pltpu.sync_copy(dst2.at[pl.ds(r0, 32)], didx_v)
        pltpu.sync_copy(w_hbm.at[2 * layer, pl.ds(r0, 32)], w0b_v)
        pltpu.sync_copy(w_hbm.at[2 * layer + 1, pl.ds(r0, 32)], w1b_v)

        def row(j, _2):
            pltpu.async_copy(h_hbm.at[sidx_v.at[j]], hs_v, sem).wait()
            for c in range(5):
                sl = pl.ds(c * 16, 16)
                dl = didx_v[j, sl] - base_node
                zi = jnp.where(
                    jnp.logical_and(dl >= 0, dl < _HALF), dl, _HALF)
                zi_v[sl] = zi
                zi1_v[sl] = zi + _SLAB

            def edge(r, _3):
                ridx = jnp.full((16,), r, _i32)
                jidx = jnp.full((16,), j, _i32)
                w0b = plsc.load_gather(w0b_v, [jidx, ridx])
                w1b = plsc.load_gather(w1b_v, [jidx, ridx])
                for c in range(8):
                    sl = pl.ds(c * 16, 16)
                    hs = hs_v[r, sl]
                    m0_v[r, sl] = hs * w0b
                    m1_v[r, sl] = hs * w1b
                return 0

            lax.fori_loop(0, _B, edge, 0)
            pltpu.sync_copy(m0_v, acc_sp.at[zi_v], add=True)
            pltpu.sync_copy(m1_v, acc_sp.at[zi1_v], add=True)
            return 0

        lax.fori_loop(0, 32, row, 0)
        return 0

    lax.fori_loop(0, rows_per_tile // 32, group, 0)
    plsc.subcore_barrier()

    r0 = sid * 312
    pltpu.sync_copy(acc_sp.at[pl.ds(r0, 312)],
                    z0_out.at[pl.ds(base_node + r0, 312)])
    pltpu.sync_copy(acc_sp.at[pl.ds(_SLAB + r0, 312)],
                    z1_out.at[pl.ds(base_node + r0, 312)])

    @pl.when(sid == 0)
    def _():
        pltpu.sync_copy(acc_sp.at[pl.ds(4992, 8)],
                        z0_out.at[pl.ds(base_node + 4992, 8)])
        pltpu.sync_copy(acc_sp.at[pl.ds(_SLAB + 4992, 8)],
                        z1_out.at[pl.ds(base_node + 4992, 8)])


def _z_pass(layer, h, src2, dst2, w, zer2d):
    kfn = pl.kernel(
        functools.partial(_z_body, layer),
        out_type=(jax.ShapeDtypeStruct((_N, _D), _f32),
                  jax.ShapeDtypeStruct((_N, _D), _f32)),
        scratch_types=[
            pltpu.VMEM_SHARED((2 * _SLAB, _D), _f32),
            pltpu.VMEM((32, _B), _i32),
            pltpu.VMEM((32, _B), _i32),
            pltpu.VMEM((32, _B), _f32),
            pltpu.VMEM((32, _B), _f32),
            pltpu.VMEM((_B,), _i32),
            pltpu.VMEM((_B,), _i32),
            pltpu.VMEM((_B, _D), _f32),
            pltpu.VMEM((_B, _D), _f32),
            pltpu.VMEM((_B, _D), _f32),
            pltpu.SemaphoreType.DMA,
        ],
        **_SC_PARAMS,
    )
    return kfn(h, src2, dst2, w, zer2d)


def _num_body(h_hbm, src2, dst2, zer2d,
              num_out,
              acc_sp,
              sidx_v, didx_v, si_v,
              hs_v, hd_v, sem, sem2):
    cid = lax.axis_index("c")
    sid = lax.axis_index("s")
    base_node = cid * _HALF

    for q in range(5):
        r0 = sid * 320 + q * 64
        pltpu.sync_copy(zer2d.at[pl.ds(0, 64)], acc_sp.at[pl.ds(r0, 64)])
    plsc.subcore_barrier()

    rows_per_tile = _ROWS // 16

    def group(g, _):
        r0 = sid * rows_per_tile + g * 32
        pltpu.sync_copy(src2.at[pl.ds(r0, 32)], sidx_v)
        pltpu.sync_copy(dst2.at[pl.ds(r0, 32)], didx_v)

        def row(j, _2):
            cp1 = pltpu.async_copy(h_hbm.at[sidx_v.at[j]], hs_v, sem)
            cp2 = pltpu.async_copy(h_hbm.at[didx_v.at[j]], hd_v, sem2)
            cp1.wait()
            cp2.wait()
            for c in range(5):
                sl = pl.ds(c * 16, 16)
                sl2 = sidx_v[j, sl] - base_node
                si_v[sl] = jnp.where(
                    jnp.logical_and(sl2 >= 0, sl2 < _HALF), sl2, _HALF)

            def edge(r, _3):
                for c in range(8):
                    sl = pl.ds(c * 16, 16)
                    df = hs_v[r, sl] - hd_v[r, sl]
                    hd_v[r, sl] = df * df
                return 0

            lax.fori_loop(0, _B, edge, 0)
            pltpu.sync_copy(hd_v, acc_sp.at[si_v], add=True)
            return 0

        lax.fori_loop(0, 32, row, 0)
        return 0

    lax.fori_loop(0, rows_per_tile // 32, group, 0)
    plsc.subcore_barrier()

    r0 = sid * 312
    pltpu.sync_copy(acc_sp.at[pl.ds(r0, 312)],
                    num_out.at[pl.ds(base_node + r0, 312)])

    @pl.when(sid == 0)
    def _():
        pltpu.sync_copy(acc_sp.at[pl.ds(4992, 8)],
                        num_out.at[pl.ds(base_node + 4992, 8)])


def _num_pass(h, src2, dst2, zer2d):
    kfn = pl.kernel(
        _num_body,
        out_type=jax.ShapeDtypeStruct((_N, _D), _f32),
        scratch_types=[
            pltpu.VMEM_SHARED((_SLAB, _D), _f32),
            pltpu.VMEM((32, _B), _i32),
            pltpu.VMEM((32, _B), _i32),
            pltpu.VMEM((_B,), _i32),
            pltpu.VMEM((_B, _D), _f32),
            pltpu.VMEM((_B, _D), _f32),
            pltpu.SemaphoreType.DMA,
            pltpu.SemaphoreType.DMA,
        ],
        **_SC_PARAMS,
    )
    return kfn(h, src2, dst2, zer2d)


_MB = 2000


def _dense_in(x, W, b2):
    def body(x_ref, w_ref, b_ref, o_ref):
        o_ref[...] = jnp.maximum(
            jnp.dot(x_ref[...], w_ref[...], preferred_element_type=_f32)
            + b_ref[...], 0.0)

    return pl.pallas_call(
        body,
        grid=(_N // _MB,),
        in_specs=[
            pl.BlockSpec((_MB, _D), lambda i: (i, 0)),
            pl.BlockSpec((_D, _D), lambda i: (0, 0)),
            pl.BlockSpec((1, _D), lambda i: (0, 0)),
        ],
        out_specs=pl.BlockSpec((_MB, _D), lambda i: (i, 0)),
        out_shape=jax.ShapeDtypeStruct((_N, _D), _f32),
    )(x, W, b2)


def _gate(h, z0, z1, num, deg2, root, bas0, bas1, bias2, Wout, bout2, final):
    def body(h_ref, z0_ref, z1_ref, num_ref, deg_ref, root_ref, b0_ref,
             b1_ref, bias_ref, wo_ref, bo_ref, hn_ref, o_ref):
        hv = h_ref[...]
        xc = (jnp.dot(hv, root_ref[...], preferred_element_type=_f32)
              + jnp.dot(z0_ref[...], b0_ref[...], preferred_element_type=_f32)
              + jnp.dot(z1_ref[...], b1_ref[...], preferred_element_type=_f32)
              + bias_ref[...])
        x_ = jnp.maximum(xc, 0.0)
        tau = jnp.tanh(num_ref[...] / jnp.maximum(deg_ref[...], 1.0))
        hn = (1.0 - tau) * hv + tau * x_
        hn_ref[...] = hn
        if final:
            o_ref[...] = (jnp.dot(hn, wo_ref[...],
                                  preferred_element_type=_f32) + bo_ref[...])
        else:
            o_ref[...] = jnp.zeros_like(o_ref)

    mat = pl.BlockSpec((_MB, _D), lambda i: (i, 0))
    w128 = pl.BlockSpec((_D, _D), lambda i: (0, 0))
    row1 = pl.BlockSpec((1, _D), lambda i: (0, 0))
    return pl.pallas_call(
        body,
        grid=(_N // _MB,),
        in_specs=[mat, mat, mat, mat,
                  pl.BlockSpec((_MB, 1), lambda i: (i, 0)),
                  w128, w128, w128, row1, w128, row1],
        out_specs=(mat, mat),
        out_shape=(jax.ShapeDtypeStruct((_N, _D), _f32),
                   jax.ShapeDtypeStruct((_N, _D), _f32)),
    )(h, z0, z1, num, deg2, root, bas0, bas1, bias2, Wout, bout2)


def kernel(x, edge_index, edge_type, W_in, b_in, rgcn_bases, rgcn_comp,
           rgcn_root, rgcn_bias, gg_bases, gg_comp, gg_root, gg_bias,
           W_out, b_out):
    pad = _EP - _E
    src2 = jnp.concatenate(
        [edge_index[0], jnp.zeros((pad,), _i32)]).reshape(_ROWS, _B)
    dst2 = jnp.concatenate(
        [edge_index[1], jnp.zeros((pad,), _i32)]).reshape(_ROWS, _B)
    et2 = jnp.concatenate(
        [edge_type, jnp.full((pad,), _R, _i32)]).reshape(_ROWS, _B)
    compt = jnp.concatenate(
        [jnp.transpose(rgcn_comp, (0, 2, 1)).reshape(_L * _NB, _R),
         jnp.zeros((_L * _NB, 8 - _R), _f32)], axis=1).reshape(_L * _NB * 8)
    zer1d = jnp.zeros((_CNT_PAD // 16,), _f32)
    ones1d = jnp.ones((_B,), _f32)
    zer2d = jnp.zeros((64, _D), _f32)

    w, deg = _graph_stats(src2, dst2, et2, compt, zer1d, ones1d)
    deg2 = deg[:_N].reshape(_N, 1)

    h = _dense_in(x, W_in, b_in.reshape(1, _D))
    latents = [h]
    out = None
    for l in range(_L):
        z0, z1 = _z_pass(l, h, src2, dst2, w, zer2d)
        num = _num_pass(h, src2, dst2, zer2d)
        h, out = _gate(h, z0, z1, num, deg2,
                       rgcn_root[l], rgcn_bases[l, 0], rgcn_bases[l, 1],
                       rgcn_bias[l].reshape(1, _D),
                       W_out, b_out.reshape(1, _D), final=(l == _L - 1))
        latents.append(h)
    return (out, *latents)


# speedup vs baseline: 2.5942x; 1.5372x over previous
import functools

import jax
import jax.numpy as jnp
from jax import lax
from jax.experimental import pallas as pl
from jax.experimental.pallas import tpu as pltpu
from jax.experimental.pallas import tpu_sc as plsc

_N = 10000
_D = 128
_E = 320000
_R = 4
_NB = 2
_L = 2

_B = 80
_ROWS = 4096
_BE = 32
_SLABZ = 5056
_ROWSE = 10240
_WROW = _BE * _D
_EP = _ROWS * _B
_HALF = _N // 2
_SLAB = 5120
_CNT_PAD = 40960
_DEG_PAD = 10240

_f32 = jnp.float32
_i32 = jnp.int32

_SC_PARAMS = dict(
    mesh=plsc.VectorSubcoreMesh(core_axis_name="c", subcore_axis_name="s"),
    compiler_params=pltpu.CompilerParams(needs_layout_passes=False),
)


def _stats_body(src2, dst2, et2, compt, zer1d, ones1d,
                w_out, deg_out,
                cnt_sp, deg_sp,
                src_v, dst_v, et_v, idx_v, didx_v, ones_v, cnt_v, comp_v,
                w0_v, w1_v, w2_v, w3_v, csem):
    cid = lax.axis_index("c")
    sid = lax.axis_index("s")

    pltpu.sync_copy(zer1d.at[pl.ds(0, _CNT_PAD // 16)],
                    cnt_sp.at[pl.ds(sid * (_CNT_PAD // 16), _CNT_PAD // 16)])
    pltpu.sync_copy(zer1d.at[pl.ds(0, _DEG_PAD // 16)],
                    deg_sp.at[pl.ds(sid * (_DEG_PAD // 16), _DEG_PAD // 16)])
    pltpu.sync_copy(ones1d, ones_v)
    pltpu.sync_copy(compt, comp_v)
    plsc.subcore_barrier()

    rows_per_tile = _ROWS // 16

    def count_block(blk, _):
        r0 = sid * rows_per_tile + blk * 32
        pltpu.sync_copy(src2.at[pl.ds(r0, 32)], src_v)
        pltpu.sync_copy(dst2.at[pl.ds(r0, 32)], dst_v)
        pltpu.sync_copy(et2.at[pl.ds(r0, 32)], et_v)

        def compute_idx(j, _2):
            for c in range(5):
                sl = pl.ds(c * 16, 16)
                t16 = et_v[j, sl]
                idx_v[j, sl] = t16 * _N + dst_v[j, sl]
                didx_v[j, sl] = jnp.where(t16 >= _R, _N, src_v[j, sl])
            return 0

        lax.fori_loop(0, 32, compute_idx, 0)

        def scatter_row(j, _2):
            pltpu.async_copy(ones_v, cnt_sp.at[idx_v.at[j]], csem, add=True)
            pltpu.async_copy(ones_v, deg_sp.at[didx_v.at[j]], csem, add=True)
            return 0

        lax.fori_loop(0, 32, scatter_row, 0)

        def drain_row(j, _2):
            pltpu.make_async_copy(ones_v, cnt_sp.at[idx_v.at[j]], csem).wait()
            pltpu.make_async_copy(
                ones_v, deg_sp.at[didx_v.at[j]], csem).wait()
            return 0

        lax.fori_loop(0, 32, drain_row, 0)
        return 0

    lax.fori_loop(0, rows_per_tile // 32, count_block, 0)
    plsc.subcore_barrier()

    @pl.when(jnp.logical_and(cid == 0, sid == 0))
    def _():
        pltpu.sync_copy(deg_sp, deg_out)

    pltpu.sync_copy(cnt_sp, cnt_v)
    wid = cid * 16 + sid
    rows_per_w = _ROWS // 32

    def weight_block(blk, _):
        r0 = wid * rows_per_w + blk * 32
        pltpu.sync_copy(dst2.at[pl.ds(r0, 32)], dst_v)
        pltpu.sync_copy(et2.at[pl.ds(r0, 32)], et_v)
        wbufs = (w0_v, w1_v, w2_v, w3_v)

        def compute_w(j, _2):
            for c in range(5):
                sl = pl.ds(c * 16, 16)
                t16 = et_v[j, sl]
                i4 = t16 * _N + dst_v[j, sl]
                ic = 1.0 / plsc.load_gather(cnt_v, [i4])
                for lb in range(4):
                    cw = plsc.load_gather(comp_v, [t16 + (lb * 8)])
                    wbufs[lb][j, sl] = cw * ic
            return 0

        lax.fori_loop(0, 32, compute_w, 0)
        for lb in range(4):
            pltpu.sync_copy(wbufs[lb], w_out.at[lb, pl.ds(r0, 32)])
        return 0

    lax.fori_loop(0, rows_per_w // 32, weight_block, 0)


def _graph_stats(src2, dst2, et2, compt, zer1d, ones1d):
    kfn = pl.kernel(
        _stats_body,
        out_type=(jax.ShapeDtypeStruct((4, _ROWS, _B), _f32),
                  jax.ShapeDtypeStruct((_DEG_PAD,), _f32)),
        scratch_types=[
            pltpu.VMEM_SHARED((_CNT_PAD,), _f32),
            pltpu.VMEM_SHARED((_DEG_PAD,), _f32),
            pltpu.VMEM((32, _B), _i32),
            pltpu.VMEM((32, _B), _i32),
            pltpu.VMEM((32, _B), _i32),
            pltpu.VMEM((32, _B), _i32),
            pltpu.VMEM((32, _B), _i32),
            pltpu.VMEM((_B,), _f32),
            pltpu.VMEM((_CNT_PAD,), _f32),
            pltpu.VMEM((32,), _f32),
            pltpu.VMEM((32, _B), _f32),
            pltpu.VMEM((32, _B), _f32),
            pltpu.VMEM((32, _B), _f32),
            pltpu.VMEM((32, _B), _f32),
            pltpu.SemaphoreType.DMA,
        ],
        **_SC_PARAMS,
    )
    return kfn(src2, dst2, et2, compt, zer1d, ones1d)


def _drain(sem, n, h_hbm, zbuf_v):
    for _ in range(n):
        pltpu.make_async_copy(h_hbm.at[pl.ds(0, 8)], zbuf_v, sem).wait()


def _z_body(layer, h_hbm, src2, dst2, w_hbm,
            z0_out, z1_out,
            acc_sp,
            zbuf_v,
            sidx_v, didx_v, w0b_v, w1b_v,
            ziA_v, zi1A_v, ziB_v, zi1B_v,
            hsA_v, hsB_v, m0A_v, m1A_v, m0B_v, m1B_v,
            gsA, gsB, ssA, ssB):
    cid = lax.axis_index("c")
    sid = lax.axis_index("s")
    base_node = cid * _HALF

    def zfill(k, _):
        zbuf_v[k // 8, pl.ds((k % 8) * 16, 16)] = jnp.zeros((16,), _f32)
        return 0

    lax.fori_loop(0, 64, zfill, 0)

    def zero_blk(q, _):
        r0 = sid * 632 + q * 8
        pltpu.sync_copy(zbuf_v, acc_sp.at[pl.ds(r0, 8)])
        return 0

    lax.fori_loop(0, 79, zero_blk, 0)
    plsc.subcore_barrier()

    rows_per_tile = _ROWSE // 16

    def compute_row(j, hs_v, m0_v, m1_v, zi_v, zi1_v):
        for c in range(2):
            sl = pl.ds(c * 16, 16)
            dl = didx_v[j, sl] - base_node
            zi = jnp.where(jnp.logical_and(dl >= 0, dl < _HALF), dl, _HALF)
            zi_v[sl] = zi
            zi1_v[sl] = zi + _SLABZ

        def edge(r, _3):
            ridx = jnp.full((16,), r, _i32)
            jidx = jnp.full((16,), j, _i32)
            w0b = plsc.load_gather(w0b_v, [jidx, ridx])
            w1b = plsc.load_gather(w1b_v, [jidx, ridx])
            for c in range(8):
                sl = pl.ds(c * 16, 16)
                hs = hs_v[r, sl]
                m0_v[r, sl] = hs * w0b
                m1_v[r, sl] = hs * w1b
            return 0

        lax.fori_loop(0, _BE, edge, 0)

    def group(g, _):
        r0 = sid * rows_per_tile + g * 32
        pltpu.sync_copy(src2.at[pl.ds(r0, 32)], sidx_v)
        pltpu.sync_copy(dst2.at[pl.ds(r0, 32)], didx_v)
        pltpu.sync_copy(w_hbm.at[2 * layer, pl.ds(r0, 32)], w0b_v)
        pltpu.sync_copy(w_hbm.at[2 * layer + 1, pl.ds(r0, 32)], w1b_v)
        pltpu.async_copy(h_hbm.at[sidx_v.at[0]], hsA_v, gsA)

        def pair(j2, _2):
            a = 2 * j2
            pltpu.async_copy(h_hbm.at[sidx_v.at[a + 1]], hsB_v, gsB)
            _drain(gsA, 4, h_hbm, zbuf_v)

            @pl.when(j2 > 0)
            def _():
                _drain(ssA, 8, h_hbm, zbuf_v)

            compute_row(a, hsA_v, m0A_v, m1A_v, ziA_v, zi1A_v)
            pltpu.async_copy(m0A_v, acc_sp.at[ziA_v], ssA, add=True)
            pltpu.async_copy(m1A_v, acc_sp.at[zi1A_v], ssA, add=True)

            @pl.when(j2 < 15)
            def _():
                pltpu.async_copy(h_hbm.at[sidx_v.at[a + 2]], hsA_v, gsA)

            _drain(gsB, 4, h_hbm, zbuf_v)

            @pl.when(j2 > 0)
            def _():
                _drain(ssB, 8, h_hbm, zbuf_v)

            compute_row(a + 1, hsB_v, m0B_v, m1B_v, ziB_v, zi1B_v)
            pltpu.async_copy(m0B_v, acc_sp.at[ziB_v], ssB, add=True)
            pltpu.async_copy(m1B_v, acc_sp.at[zi1B_v], ssB, add=True)
            return 0

        lax.fori_loop(0, 16, pair, 0)
        _drain(ssA, 8, h_hbm, zbuf_v)
        _drain(ssB, 8, h_hbm, zbuf_v)
        return 0

    lax.fori_loop(0, rows_per_tile // 32, group, 0)
    plsc.subcore_barrier()

    r0 = sid * 312
    pltpu.sync_copy(acc_sp.at[pl.ds(r0, 312)],
                    z0_out.at[pl.ds(base_node + r0, 312)])
    pltpu.sync_copy(acc_sp.at[pl.ds(_SLABZ + r0, 312)],
                    z1_out.at[pl.ds(base_node + r0, 312)])

    @pl.when(sid == 0)
    def _():
        pltpu.sync_copy(acc_sp.at[pl.ds(4992, 8)],
                        z0_out.at[pl.ds(base_node + 4992, 8)])
        pltpu.sync_copy(acc_sp.at[pl.ds(_SLABZ + 4992, 8)],
                        z1_out.at[pl.ds(base_node + 4992, 8)])


def _z_pass(layer, h, src2, dst2, w):
    kfn = pl.kernel(
        functools.partial(_z_body, layer),
        out_type=(jax.ShapeDtypeStruct((_N, _D), _f32),
                  jax.ShapeDtypeStruct((_N, _D), _f32)),
        scratch_types=[
            pltpu.VMEM_SHARED((2 * _SLABZ, _D), _f32),
            pltpu.VMEM((8, _D), _f32),
            pltpu.VMEM((32, _BE), _i32),
            pltpu.VMEM((32, _BE), _i32),
            pltpu.VMEM((32, _BE), _f32),
            pltpu.VMEM((32, _BE), _f32),
            pltpu.VMEM((_BE,), _i32),
            pltpu.VMEM((_BE,), _i32),
            pltpu.VMEM((_BE,), _i32),
            pltpu.VMEM((_BE,), _i32),
            pltpu.VMEM((_BE, _D), _f32),
            pltpu.VMEM((_BE, _D), _f32),
            pltpu.VMEM((_BE, _D), _f32),
            pltpu.VMEM((_BE, _D), _f32),
            pltpu.VMEM((_BE, _D), _f32),
            pltpu.VMEM((_BE, _D), _f32),
            pltpu.SemaphoreType.DMA,
            pltpu.SemaphoreType.DMA,
            pltpu.SemaphoreType.DMA,
            pltpu.SemaphoreType.DMA,
        ],
        **_SC_PARAMS,
    )
    return kfn(h, src2, dst2, w)


def _num_body(h_hbm, gidx2, skey2,
              s1_out, s2_out,
              acc_sp,
              zbuf_v,
              gidx_v, skey_v,
              siA_v, si2A_v, siB_v, si2B_v,
              hdA_v, hdB_v, m0A_v, m1A_v, m0B_v, m1B_v,
              gsA, gsB, ssA, ssB):
    cid = lax.axis_index("c")
    sid = lax.axis_index("s")
    base_node = cid * _HALF

    def zfill(k, _):
        zbuf_v[k // 8, pl.ds((k % 8) * 16, 16)] = jnp.zeros((16,), _f32)
        return 0

    lax.fori_loop(0, 64, zfill, 0)

    def zero_blk(q, _):
        r0 = sid * 632 + q * 8
        pltpu.sync_copy(zbuf_v, acc_sp.at[pl.ds(r0, 8)])
        return 0

    lax.fori_loop(0, 79, zero_blk, 0)
    plsc.subcore_barrier()

    rows_per_tile = _ROWSE // 16

    def compute_row(j, hd_v, m0_v, m1_v, si_v, si2_v):
        for c in range(2):
            sl = pl.ds(c * 16, 16)
            sl2 = skey_v[j, sl] - base_node
            si = jnp.where(jnp.logical_and(sl2 >= 0, sl2 < _HALF),
                           sl2, _HALF)
            si_v[sl] = si
            si2_v[sl] = si + _SLABZ

        def edge(r, _3):
            for c in range(8):
                sl = pl.ds(c * 16, 16)
                hd = hd_v[r, sl]
                m0_v[r, sl] = hd
                m1_v[r, sl] = hd * hd
            return 0

        lax.fori_loop(0, _BE, edge, 0)

    def group(g, _):
        r0 = sid * rows_per_tile + g * 32
        pltpu.sync_copy(gidx2.at[pl.ds(r0, 32)], gidx_v)
        pltpu.sync_copy(skey2.at[pl.ds(r0, 32)], skey_v)
        pltpu.async_copy(h_hbm.at[gidx_v.at[0]], hdA_v, gsA)

        def pair(j2, _2):
            a = 2 * j2
            pltpu.async_copy(h_hbm.at[gidx_v.at[a + 1]], hdB_v, gsB)
            _drain(gsA, 4, h_hbm, zbuf_v)

            @pl.when(j2 > 0)
            def _():
                _drain(ssA, 8, h_hbm, zbuf_v)

            compute_row(a, hdA_v, m0A_v, m1A_v, siA_v, si2A_v)
            pltpu.async_copy(m0A_v, acc_sp.at[siA_v], ssA, add=True)
            pltpu.async_copy(m1A_v, acc_sp.at[si2A_v], ssA, add=True)

            @pl.when(j2 < 15)
            def _():
                pltpu.async_copy(h_hbm.at[gidx_v.at[a + 2]], hdA_v, gsA)

            _drain(gsB, 4, h_hbm, zbuf_v)

            @pl.when(j2 > 0)
            def _():
                _drain(ssB, 8, h_hbm, zbuf_v)

            compute_row(a + 1, hdB_v, m0B_v, m1B_v, siB_v, si2B_v)
            pltpu.async_copy(m0B_v, acc_sp.at[siB_v], ssB, add=True)
            pltpu.async_copy(m1B_v, acc_sp.at[si2B_v], ssB, add=True)
            return 0

        lax.fori_loop(0, 16, pair, 0)
        _drain(ssA, 8, h_hbm, zbuf_v)
        _drain(ssB, 8, h_hbm, zbuf_v)
        return 0

    lax.fori_loop(0, rows_per_tile // 32, group, 0)
    plsc.subcore_barrier()

    r0 = sid * 312
    pltpu.sync_copy(acc_sp.at[pl.ds(r0, 312)],
                    s1_out.at[pl.ds(base_node + r0, 312)])
    pltpu.sync_copy(acc_sp.at[pl.ds(_SLABZ + r0, 312)],
                    s2_out.at[pl.ds(base_node + r0, 312)])

    @pl.when(sid == 0)
    def _():
        pltpu.sync_copy(acc_sp.at[pl.ds(4992, 8)],
                        s1_out.at[pl.ds(base_node + 4992, 8)])
        pltpu.sync_copy(acc_sp.at[pl.ds(_SLABZ + 4992, 8)],
                        s2_out.at[pl.ds(base_node + 4992, 8)])


def _num_pass(h, gidx2, skey2):
    kfn = pl.kernel(
        _num_body,
        out_type=(jax.ShapeDtypeStruct((_N, _D), _f32),
                  jax.ShapeDtypeStruct((_N, _D), _f32)),
        scratch_types=[
            pltpu.VMEM_SHARED((2 * _SLABZ, _D), _f32),
            pltpu.VMEM((8, _D), _f32),
            pltpu.VMEM((32, _BE), _i32),
            pltpu.VMEM((32, _BE), _i32),
            pltpu.VMEM((_BE,), _i32),
            pltpu.VMEM((_BE,), _i32),
            pltpu.VMEM((_BE,), _i32),
            pltpu.VMEM((_BE,), _i32),
            pltpu.VMEM((_BE, _D), _f32),
            pltpu.VMEM((_BE, _D), _f32),
            pltpu.VMEM((_BE, _D), _f32),
            pltpu.VMEM((_BE, _D), _f32),
            pltpu.VMEM((_BE, _D), _f32),
            pltpu.VMEM((_BE, _D), _f32),
            pltpu.SemaphoreType.DMA,
            pltpu.SemaphoreType.DMA,
            pltpu.SemaphoreType.DMA,
            pltpu.SemaphoreType.DMA,
        ],
        **_SC_PARAMS,
    )
    return kfn(h, gidx2, skey2)


_MB = 2000


def _dense_in(x, W, b2):
    def body(x_ref, w_ref, b_ref, o_ref):
        o_ref[...] = jnp.maximum(
            jnp.dot(x_ref[...], w_ref[...], preferred_element_type=_f32)
            + b_ref[...], 0.0)

    return pl.pallas_call(
        body,
        grid=(_N // _MB,),
        in_specs=[
            pl.BlockSpec((_MB, _D), lambda i: (i, 0)),
            pl.BlockSpec((_D, _D), lambda i: (0, 0)),
            pl.BlockSpec((1, _D), lambda i: (0, 0)),
        ],
        out_specs=pl.BlockSpec((_MB, _D), lambda i: (i, 0)),
        out_shape=jax.ShapeDtypeStruct((_N, _D), _f32),
    )(x, W, b2)


def _gate(h, z0, z1, s1, s2, deg2, root, bas0, bas1, bias2, Wout, bout2,
          final):
    def body(h_ref, z0_ref, z1_ref, s1_ref, s2_ref, deg_ref, root_ref,
             b0_ref, b1_ref, bias_ref, wo_ref, bo_ref, hn_ref, o_ref):
        hv = h_ref[...]
        xc = (jnp.dot(hv, root_ref[...], preferred_element_type=_f32)
              + jnp.dot(z0_ref[...], b0_ref[...], preferred_element_type=_f32)
              + jnp.dot(z1_ref[...], b1_ref[...], preferred_element_type=_f32)
              + bias_ref[...])
        x_ = jnp.maximum(xc, 0.0)
        dv = deg_ref[...]
        num = dv * hv * hv - 2.0 * hv * s1_ref[...] + s2_ref[...]
        tau = jnp.tanh(num / jnp.maximum(dv, 1.0))
        hn = (1.0 - tau) * hv + tau * x_
        hn_ref[...] = hn
        if final:
            o_ref[...] = (jnp.dot(hn, wo_ref[...],
                                  preferred_element_type=_f32) + bo_ref[...])
        else:
            o_ref[...] = jnp.zeros_like(o_ref)

    mat = pl.BlockSpec((_MB, _D), lambda i: (i, 0))
    w128 = pl.BlockSpec((_D, _D), lambda i: (0, 0))
    row1 = pl.BlockSpec((1, _D), lambda i: (0, 0))
    return pl.pallas_call(
        body,
        grid=(_N // _MB,),
        in_specs=[mat, mat, mat, mat, mat,
                  pl.BlockSpec((_MB, 1), lambda i: (i, 0)),
                  w128, w128, w128, row1, w128, row1],
        out_specs=(mat, mat),
        out_shape=(jax.ShapeDtypeStruct((_N, _D), _f32),
                   jax.ShapeDtypeStruct((_N, _D), _f32)),
    )(h, z0, z1, s1, s2, deg2, root, bas0, bas1, bias2, Wout, bout2)


def kernel(x, edge_index, edge_type, W_in, b_in, rgcn_bases, rgcn_comp,
           rgcn_root, rgcn_bias, gg_bases, gg_comp, gg_root, gg_bias,
           W_out, b_out):
    pad = _EP - _E
    src2 = jnp.concatenate(
        [edge_index[0], jnp.zeros((pad,), _i32)]).reshape(_ROWS, _B)
    dst2 = jnp.concatenate(
        [edge_index[1], jnp.zeros((pad,), _i32)]).reshape(_ROWS, _B)
    et2 = jnp.concatenate(
        [edge_type, jnp.full((pad,), _R, _i32)]).reshape(_ROWS, _B)
    compt = jnp.concatenate(
        [jnp.transpose(rgcn_comp, (0, 2, 1)).reshape(_L * _NB, _R),
         jnp.zeros((_L * _NB, 8 - _R), _f32)], axis=1).reshape(_L * _NB * 8)
    zer1d = jnp.zeros((_CNT_PAD // 16,), _f32)
    ones1d = jnp.ones((_B,), _f32)

    w, deg = _graph_stats(src2, dst2, et2, compt, zer1d, ones1d)
    src3 = src2.reshape(_ROWSE, _BE)
    dst3 = dst2.reshape(_ROWSE, _BE)
    w3 = w.reshape(4, _ROWSE, _BE)
    deg2 = deg[:_N].reshape(_N, 1)
    skey3 = jnp.concatenate(
        [edge_index[0], jnp.full((pad,), 1 << 20, _i32)]).reshape(_ROWSE, _BE)

    h = _dense_in(x, W_in, b_in.reshape(1, _D))
    latents = [h]
    out = None
    for l in range(_L):
        z0, z1 = _z_pass(l, h, src3, dst3, w3)
        s1, s2 = _num_pass(h, dst3, skey3)
        h, out = _gate(h, z0, z1, s1, s2, deg2,
                       rgcn_root[l], rgcn_bases[l, 0], rgcn_bases[l, 1],
                       rgcn_bias[l].reshape(1, _D),
                       W_out, b_out.reshape(1, _D), final=(l == _L - 1))
        latents.append(h)
    return (out, *latents)
